# Initial kernel scaffold; baseline (speedup 1.0000x reference)
import jax
import jax.numpy as jnp
import numpy as np
from jax.experimental import pallas as pl
from jax.experimental.pallas import tpu as pltpu

_BATCH = 128
_F = 2048
_OUT = 2048
_CMS = 32
_U = 64
_TILE = 256
_GRID = _OUT // _TILE


def _gumbel_const():
    cpu = jax.devices("cpu")[0]
    with jax.default_device(cpu):
        base = jax.random.key(42)
        gs = [
            jax.random.gumbel(jax.random.fold_in(base, i), (_BATCH, _U), jnp.float32)
            for i in range(_CMS)
        ]
        g = jnp.stack(gs, axis=0)
        gt = jnp.transpose(g, (0, 2, 1)).reshape(_OUT, _BATCH)
        return np.asarray(gt)


_GT = _gumbel_const()


def _body(xt_ref, w_ref, b_ref, g_ref, out_ref, zs_ref):
    k = pl.program_id(0)
    xt = xt_ref[...]
    zt = jax.lax.dot_general(
        w_ref[...], xt, (((1,), (0,)), ((), ())),
        preferred_element_type=jnp.float32)
    na = jnp.sum(xt, axis=0, keepdims=True)
    zs_ref[pl.ds(k * _TILE, _TILE), :] = (zt + b_ref[...]) / na

    @pl.when(k == _GRID - 1)
    def _sample():
        z = zs_ref[...]
        m = jnp.max(z, axis=0, keepdims=True)
        row_cm = jax.lax.broadcasted_iota(jnp.int32, (_OUT, _BATCH), 0) // _U
        lane = jax.lax.broadcasted_iota(jnp.int32, (_OUT, _BATCH), 1)
        sel = (row_cm == lane).astype(jnp.float32)
        mrows = jax.lax.dot_general(
            sel, m, (((1,), (1,)), ((), ())),
            preferred_element_type=jnp.float32)
        scores = jnp.exp(z - mrows) + g_ref[...]
        s3 = scores.reshape(_CMS, _U, _BATCH)
        gm = jnp.max(s3, axis=1, keepdims=True)
        sub = jax.lax.broadcasted_iota(jnp.int32, (_CMS, _U, _BATCH), 1)
        idx = jnp.min(jnp.where(s3 >= gm, sub, _U), axis=1, keepdims=True)
        out_ref[...] = (sub == idx).astype(jnp.float32).reshape(_OUT, _BATCH)


@jax.jit
def _impl(x, W_in, b_in):
    xt = x.T
    b2 = b_in.reshape(_OUT, 1)
    g = jnp.asarray(_GT)
    out_t = pl.pallas_call(
        _body,
        grid=(_GRID,),
        in_specs=[
            pl.BlockSpec((_F, _BATCH), lambda k: (0, 0)),
            pl.BlockSpec((_TILE, _F), lambda k: (k, 0)),
            pl.BlockSpec((_TILE, 1), lambda k: (k, 0)),
            pl.BlockSpec((_OUT, _BATCH), lambda k: (0, 0)),
        ],
        out_specs=pl.BlockSpec((_OUT, _BATCH), lambda k: (0, 0)),
        out_shape=jax.ShapeDtypeStruct((_OUT, _BATCH), jnp.float32),
        scratch_shapes=[pltpu.VMEM((_OUT, _BATCH), jnp.float32)],
    )(xt, W_in, b2, g)
    return out_t.T


def kernel(x, W_in, b_in):
    return _impl(x, W_in, b_in)


# speedup vs baseline: 2.8838x; 2.8838x over previous
import jax
import jax.numpy as jnp
import numpy as np
from jax.experimental import pallas as pl
from jax.experimental.pallas import tpu as pltpu

_BATCH = 128
_F = 2048
_OUT = 2048
_CMS = 32
_U = 64
_TILE = 256
_GRID = _OUT // _TILE


def _gumbel_traced():
    base = jax.random.key(42)
    gs = [
        jax.random.gumbel(jax.random.fold_in(base, i), (_BATCH, _U), jnp.float32)
        for i in range(_CMS)
    ]
    g = jnp.stack(gs, axis=0)
    return jnp.transpose(g, (0, 2, 1)).reshape(_OUT, _BATCH)


def _body(xt_ref, w_ref, b_ref, g_ref, out_ref, zs_ref):
    k = pl.program_id(0)
    xt = xt_ref[...]
    zt = jax.lax.dot_general(
        w_ref[...], xt, (((1,), (0,)), ((), ())),
        preferred_element_type=jnp.float32)
    na = jnp.sum(xt, axis=0, keepdims=True)
    zs_ref[pl.ds(k * _TILE, _TILE), :] = (zt + b_ref[...]) / na

    @pl.when(k == _GRID - 1)
    def _sample():
        z = zs_ref[...]
        m = jnp.max(z, axis=0, keepdims=True)
        row_cm = jax.lax.broadcasted_iota(jnp.int32, (_OUT, _BATCH), 0) // _U
        lane = jax.lax.broadcasted_iota(jnp.int32, (_OUT, _BATCH), 1)
        sel = (row_cm == lane).astype(jnp.float32)
        mrows = jax.lax.dot_general(
            sel, m, (((1,), (1,)), ((), ())),
            preferred_element_type=jnp.float32)
        scores = jnp.exp(z - mrows) + g_ref[...]
        s3 = scores.reshape(_CMS, _U, _BATCH)
        gm = jnp.max(s3, axis=1, keepdims=True)
        sub = jax.lax.broadcasted_iota(jnp.int32, (_CMS, _U, _BATCH), 1)
        idx = jnp.min(jnp.where(s3 >= gm, sub, _U), axis=1, keepdims=True)
        out_ref[...] = (sub == idx).astype(jnp.float32).reshape(_OUT, _BATCH)


@jax.jit
def _impl(x, W_in, b_in):
    xt = x.T
    b2 = b_in.reshape(_OUT, 1)
    g = _gumbel_traced()
    out_t = pl.pallas_call(
        _body,
        grid=(_GRID,),
        in_specs=[
            pl.BlockSpec((_F, _BATCH), lambda k: (0, 0)),
            pl.BlockSpec((_TILE, _F), lambda k: (k, 0)),
            pl.BlockSpec((_TILE, 1), lambda k: (k, 0)),
            pl.BlockSpec((_OUT, _BATCH), lambda k: (0, 0)),
        ],
        out_specs=pl.BlockSpec((_OUT, _BATCH), lambda k: (0, 0)),
        out_shape=jax.ShapeDtypeStruct((_OUT, _BATCH), jnp.float32),
        scratch_shapes=[pltpu.VMEM((_OUT, _BATCH), jnp.float32)],
    )(xt, W_in, b2, g)
    return out_t.T


def kernel(x, W_in, b_in):
    return _impl(x, W_in, b_in)


# speedup vs baseline: 16.7481x; 5.8077x over previous
import jax
import jax.numpy as jnp
import numpy as np
from jax.experimental import pallas as pl
from jax.experimental.pallas import tpu as pltpu

_BATCH = 128
_F = 2048
_OUT = 2048
_CMS = 32
_U = 64
_TILE = 256
_GRID = _OUT // _TILE


def _gumbel_traced():
    base = jax.random.key(42)
    keys = jax.vmap(jax.random.fold_in, in_axes=(None, 0))(base, jnp.arange(_CMS))
    g = jax.vmap(lambda k: jax.random.gumbel(k, (_BATCH, _U), jnp.float32))(keys)
    return jnp.transpose(g, (0, 2, 1))


def _body(xt_ref, w_ref, b_ref, g_ref, out_ref, zs_ref):
    k = pl.program_id(0)
    xt = xt_ref[...]
    zt = jax.lax.dot_general(
        w_ref[...], xt, (((1,), (0,)), ((), ())),
        preferred_element_type=jnp.float32)
    na = jnp.sum(xt, axis=0, keepdims=True)
    zs_ref[pl.ds(k * _TILE, _TILE), :] = (zt + b_ref[...]) / na

    @pl.when(k == _GRID - 1)
    def _sample():
        z = zs_ref[...]
        m = jnp.max(z, axis=0, keepdims=True)
        row_cm = jax.lax.broadcasted_iota(jnp.int32, (_OUT, _BATCH), 0) // _U
        lane = jax.lax.broadcasted_iota(jnp.int32, (_OUT, _BATCH), 1)
        sel = (row_cm == lane).astype(jnp.float32)
        mrows = jax.lax.dot_general(
            sel, m, (((1,), (1,)), ((), ())),
            preferred_element_type=jnp.float32)
        s3 = jnp.exp(z - mrows).reshape(_CMS, _U, _BATCH) + g_ref[...]
        gm = jnp.max(s3, axis=1, keepdims=True)
        sub = jax.lax.broadcasted_iota(jnp.int32, (_CMS, _U, _BATCH), 1)
        idx = jnp.min(jnp.where(s3 >= gm, sub, _U), axis=1, keepdims=True)
        out_ref[...] = (sub == idx).astype(jnp.float32).reshape(_OUT, _BATCH)


@jax.jit
def _impl(x, W_in, b_in):
    xt = x.T
    b2 = b_in.reshape(_OUT, 1)
    g = _gumbel_traced()
    out_t = pl.pallas_call(
        _body,
        grid=(_GRID,),
        in_specs=[
            pl.BlockSpec((_F, _BATCH), lambda k: (0, 0)),
            pl.BlockSpec((_TILE, _F), lambda k: (k, 0)),
            pl.BlockSpec((_TILE, 1), lambda k: (k, 0)),
            pl.BlockSpec((_CMS, _U, _BATCH), lambda k: (0, 0, 0)),
        ],
        out_specs=pl.BlockSpec((_OUT, _BATCH), lambda k: (0, 0)),
        out_shape=jax.ShapeDtypeStruct((_OUT, _BATCH), jnp.float32),
        scratch_shapes=[pltpu.VMEM((_OUT, _BATCH), jnp.float32)],
    )(xt, W_in, b2, g)
    return out_t.T


def kernel(x, W_in, b_in):
    return _impl(x, W_in, b_in)
